# baseline (speedup 1.0000x reference)
import functools

import jax
import jax.numpy as jnp
from jax import lax
from jax.experimental import pallas as pl
from jax.experimental.pallas import tpu as pltpu
from jax.experimental.pallas import tpu_sc as plsc

_MAX_ROWS = 8192
_EMB = 2048
_OUT_ROWS = 4096

_NC, _NS = 2, 16
_NW = _NC * _NS
_RPW = _OUT_ROWS // _NW
_CHUNK = 16
_NCHUNK = _RPW // _CHUNK

_mesh = plsc.VectorSubcoreMesh(
    core_axis_name="c", subcore_axis_name="s",
    num_cores=_NC, num_subcores=_NS,
)


@functools.partial(
    pl.kernel,
    mesh=_mesh,
    out_type=jax.ShapeDtypeStruct((_OUT_ROWS, _EMB), jnp.float32),
    scratch_types=[
        pltpu.VMEM((_NCHUNK, _CHUNK), jnp.int32),
        pltpu.VMEM((_CHUNK, _EMB), jnp.float32),
        pltpu.VMEM((_CHUNK, _EMB), jnp.float32),
        pltpu.SemaphoreType.DMA,
        pltpu.SemaphoreType.DMA,
    ],
)
def _gather_rows(table_hbm, idx_hbm, out_hbm, idx_v, buf0, buf1, sem_g, sem_s):
    wid = lax.axis_index("s") * _NC + lax.axis_index("c")
    base = wid * _RPW
    bufs = (buf0, buf1)

    pltpu.sync_copy(idx_hbm.at[wid], idx_v)

    def gather(j):
        return pltpu.async_copy(table_hbm.at[idx_v.at[j]], bufs[j % 2], sem_g)

    def scatter(j):
        return pltpu.async_copy(
            bufs[j % 2], out_hbm.at[pl.ds(base + j * _CHUNK, _CHUNK)], sem_s)

    g = gather(0)
    scatters = []
    for j in range(_NCHUNK):
        g.wait()
        scatters.append(scatter(j))
        if j + 1 < _NCHUNK:
            if j >= 1:
                scatters[j - 1].wait()
            g = gather(j + 1)
    scatters[_NCHUNK - 2].wait()
    scatters[_NCHUNK - 1].wait()


def kernel(seq_len, start_pos, pos_embeddings):
    start_row = (jnp.asarray(start_pos, jnp.int32)
                 + jnp.asarray(seq_len, jnp.int32) - _OUT_ROWS)
    start_row = jnp.clip(start_row, 0, _MAX_ROWS - _OUT_ROWS)
    row_idx = start_row + lax.iota(jnp.int32, _OUT_ROWS)
    return _gather_rows(pos_embeddings, row_idx.reshape(_NW, _NCHUNK, _CHUNK))
